# baseline (speedup 1.0000x reference)
import numpy as np
import jax
import jax.numpy as jnp
from jax.experimental import pallas as pl
from jax.experimental.pallas import tpu as pltpu

_H = 35
_W = 35
_N = _H * _W
_B = 16
_CIN = 512
_CHID = 128
_COUT = 512
_R = _N * _B
_BLK = 2800
_HALO = 560
_NSTEP = _R // _BLK


def _stencil_coeffs():
    ii, jj = np.meshgrid(np.arange(_H), np.arange(_W), indexing="ij")
    deg = 1.0 + (jj > 0) + (jj < _W - 1) + (ii > 0) + (ii < _H - 1)
    u = deg ** -0.5
    c0 = u * u
    cL = np.where(jj > 0, u * np.roll(u, 1, axis=1), 0.0)
    cR = np.where(jj < _W - 1, u * np.roll(u, -1, axis=1), 0.0)
    cU = np.where(ii > 0, u * np.roll(u, 1, axis=0), 0.0)
    cD = np.where(ii < _H - 1, u * np.roll(u, -1, axis=0), 0.0)
    r = c0 + cL + cR + cU + cD
    coef = np.zeros((_R, 8), np.float32)
    for k, v in enumerate((c0, cL, cR, cU, cD)):
        coef[:, k] = np.repeat(v.reshape(-1), _B)
    coef[:, 5] = 1.0
    coef[:, 6] = np.repeat(r.reshape(-1), _B)
    return coef


_COEF_NP = _stencil_coeffs()


def _gcn_body(x_ref, xh1_ref, xh2_ref, xh3_ref, xh4_ref,
              c_ref, ch1_ref, ch2_ref, ch3_ref, ch4_ref,
              w1_ref, w2_ref, b1_ref, b2_ref, o_ref):
    x = jnp.concatenate([xh1_ref[...], xh2_ref[...], x_ref[...],
                         xh3_ref[...], xh4_ref[...]], axis=0)
    c = jnp.concatenate([ch1_ref[...], ch2_ref[...], c_ref[...],
                         ch3_ref[...], ch4_ref[...]], axis=0)
    z = jax.lax.dot_general(x, w1_ref[...], (((1,), (0,)), ((), ())),
                            preferred_element_type=jnp.float32)

    def prop(t):
        return (c[:, 0:1] * t
                + c[:, 1:2] * pltpu.roll(t, _B, 0)
                + c[:, 2:3] * pltpu.roll(t, t.shape[0] - _B, 0)
                + c[:, 3:4] * pltpu.roll(t, _HALO, 0)
                + c[:, 4:5] * pltpu.roll(t, t.shape[0] - _HALO, 0))

    z2 = prop(prop(z))[2 * _HALO: 2 * _HALO + _BLK]
    lhs = jnp.concatenate([z2, c_ref[:, 5:7]], axis=1)
    bvec = jax.lax.dot_general(b1_ref[...], w2_ref[...], (((1,), (0,)), ((), ())),
                               preferred_element_type=jnp.float32)
    w2a = jnp.concatenate([w2_ref[...], b2_ref[...], bvec], axis=0)
    y = jax.lax.dot_general(lhs, w2a, (((1,), (0,)), ((), ())),
                            preferred_element_type=jnp.float32)
    o_ref[...] = y.astype(jnp.bfloat16)


def _halo_specs(ncols):
    u = _BLK // _HALO
    return [
        pl.BlockSpec((_BLK, ncols), lambda i: (i, 0)),
        pl.BlockSpec((_HALO, ncols), lambda i: (jnp.maximum(i * u - 2, 0), 0)),
        pl.BlockSpec((_HALO, ncols), lambda i: (jnp.maximum(i * u - 1, 0), 0)),
        pl.BlockSpec((_HALO, ncols), lambda i: (jnp.minimum(i * u + u, _R // _HALO - 1), 0)),
        pl.BlockSpec((_HALO, ncols), lambda i: (jnp.minimum(i * u + u + 1, _R // _HALO - 1), 0)),
    ]


def kernel(inFeatures, W1, b1, W2, b2):
    xin = inFeatures.transpose(2, 3, 0, 1).reshape(_R, _CIN)
    y = pl.pallas_call(
        _gcn_body,
        grid=(_NSTEP,),
        in_specs=(
            _halo_specs(_CIN)
            + _halo_specs(8)
            + [
                pl.BlockSpec((_CIN, _CHID), lambda i: (0, 0)),
                pl.BlockSpec((_CHID, _COUT), lambda i: (0, 0)),
                pl.BlockSpec((1, _CHID), lambda i: (0, 0)),
                pl.BlockSpec((1, _COUT), lambda i: (0, 0)),
            ]
        ),
        out_specs=pl.BlockSpec((_BLK, _COUT), lambda i: (i, 0)),
        out_shape=jax.ShapeDtypeStruct((_R, _COUT), jnp.bfloat16),
        compiler_params=pltpu.CompilerParams(dimension_semantics=("parallel",)),
    )(xin, xin, xin, xin, xin,
      jnp.asarray(_COEF_NP), jnp.asarray(_COEF_NP), jnp.asarray(_COEF_NP),
      jnp.asarray(_COEF_NP), jnp.asarray(_COEF_NP),
      W1, W2, b1.reshape(1, _CHID), b2.reshape(1, _COUT))
    out = y.reshape(_N, _B, _COUT).transpose(1, 0, 2).reshape(_B, _COUT, _H, _W)
    return out.astype(jnp.float32)


# speedup vs baseline: 1.0857x; 1.0857x over previous
import numpy as np
import jax
import jax.numpy as jnp
from jax.experimental import pallas as pl
from jax.experimental.pallas import tpu as pltpu

_H = 35
_W = 35
_N = _H * _W
_B = 16
_CIN = 512
_CHID = 128
_COUT = 512
_R = _N * _B
_BLK = 2800
_HALO = 560
_NSTEP = _R // _BLK
_CQ = 128


def _stencil_coeffs():
    ii, jj = np.meshgrid(np.arange(_H), np.arange(_W), indexing="ij")
    deg = 1.0 + (jj > 0) + (jj < _W - 1) + (ii > 0) + (ii < _H - 1)
    u = deg ** -0.5
    c0 = u * u
    cL = np.where(jj > 0, u * np.roll(u, 1, axis=1), 0.0)
    cR = np.where(jj < _W - 1, u * np.roll(u, -1, axis=1), 0.0)
    cU = np.where(ii > 0, u * np.roll(u, 1, axis=0), 0.0)
    cD = np.where(ii < _H - 1, u * np.roll(u, -1, axis=0), 0.0)
    r = c0 + cL + cR + cU + cD
    coef = np.zeros((_R, 8), np.float32)
    for k, v in enumerate((c0, cL, cR, cU, cD)):
        coef[:, k] = np.repeat(v.reshape(-1), _B)
    return coef, r.reshape(_N, 1).astype(np.float32)


_COEF_NP, _RCOL_NP = _stencil_coeffs()


def _stage1_body(x_ref, xh1_ref, xh2_ref, xh3_ref, xh4_ref,
                 c_ref, ch1_ref, ch2_ref, ch3_ref, ch4_ref, w1_ref, z_ref):
    x = jnp.concatenate([xh1_ref[...], xh2_ref[...], x_ref[...],
                         xh3_ref[...], xh4_ref[...]], axis=0)
    c = jnp.concatenate([ch1_ref[...], ch2_ref[...], c_ref[...],
                         ch3_ref[...], ch4_ref[...]], axis=0)
    z = jax.lax.dot_general(x, w1_ref[...], (((1,), (0,)), ((), ())),
                            preferred_element_type=jnp.float32)

    def prop(t):
        return (c[:, 0:1] * t
                + c[:, 1:2] * pltpu.roll(t, _B, 0)
                + c[:, 2:3] * pltpu.roll(t, t.shape[0] - _B, 0)
                + c[:, 3:4] * pltpu.roll(t, _HALO, 0)
                + c[:, 4:5] * pltpu.roll(t, t.shape[0] - _HALO, 0))

    z2 = prop(prop(z))[2 * _HALO: 2 * _HALO + _BLK]
    z_ref[...] = z2.reshape(_BLK // _B, _B, _CHID)


def _stage2_body(z_ref, w2_ref, b1_ref, b2_ref, r_ref, o_ref):
    bvec = jax.lax.dot_general(b1_ref[...], w2_ref[...], (((1,), (0,)), ((), ())),
                               preferred_element_type=jnp.float32)
    bias = r_ref[...] * bvec + b2_ref[...]
    for b in range(_B):
        zb = z_ref[:, b, :]
        yb = jax.lax.dot_general(zb, w2_ref[...], (((1,), (0,)), ((), ())),
                                 preferred_element_type=jnp.float32)
        o_ref[b] = yb + bias


def _halo_specs(ncols):
    u = _BLK // _HALO
    return [
        pl.BlockSpec((_BLK, ncols), lambda i: (i, 0)),
        pl.BlockSpec((_HALO, ncols), lambda i: (jnp.maximum(i * u - 2, 0), 0)),
        pl.BlockSpec((_HALO, ncols), lambda i: (jnp.maximum(i * u - 1, 0), 0)),
        pl.BlockSpec((_HALO, ncols), lambda i: (jnp.minimum(i * u + u, _R // _HALO - 1), 0)),
        pl.BlockSpec((_HALO, ncols), lambda i: (jnp.minimum(i * u + u + 1, _R // _HALO - 1), 0)),
    ]


def kernel(inFeatures, W1, b1, W2, b2):
    xin = inFeatures.transpose(2, 3, 0, 1).reshape(_R, _CIN)
    z2 = pl.pallas_call(
        _stage1_body,
        grid=(_NSTEP,),
        in_specs=(
            _halo_specs(_CIN)
            + _halo_specs(8)
            + [pl.BlockSpec((_CIN, _CHID), lambda i: (0, 0))]
        ),
        out_specs=pl.BlockSpec((_BLK // _B, _B, _CHID), lambda i: (i, 0, 0)),
        out_shape=jax.ShapeDtypeStruct((_N, _B, _CHID), jnp.float32),
        compiler_params=pltpu.CompilerParams(dimension_semantics=("parallel",)),
    )(xin, xin, xin, xin, xin,
      jnp.asarray(_COEF_NP), jnp.asarray(_COEF_NP), jnp.asarray(_COEF_NP),
      jnp.asarray(_COEF_NP), jnp.asarray(_COEF_NP), W1)

    y = pl.pallas_call(
        _stage2_body,
        grid=(_COUT // _CQ,),
        in_specs=[
            pl.BlockSpec((_N, _B, _CHID), lambda j: (0, 0, 0)),
            pl.BlockSpec((_CHID, _CQ), lambda j: (0, j)),
            pl.BlockSpec((1, _CHID), lambda j: (0, 0)),
            pl.BlockSpec((1, _CQ), lambda j: (0, j)),
            pl.BlockSpec((_N, 1), lambda j: (0, 0)),
        ],
        out_specs=pl.BlockSpec((_B, _N, _CQ), lambda j: (0, 0, j)),
        out_shape=jax.ShapeDtypeStruct((_B, _N, _COUT), jnp.float32),
        compiler_params=pltpu.CompilerParams(dimension_semantics=("parallel",)),
    )(z2, W2, b1.reshape(1, _CHID), b2.reshape(1, _COUT), jnp.asarray(_RCOL_NP))

    return y.reshape(_B, _COUT, _H, _W)
